# baseline (speedup 1.0000x reference)
import functools

import jax
import jax.numpy as jnp
from jax import lax
from jax.experimental import pallas as pl
from jax.experimental.pallas import tpu as pltpu
from jax.experimental.pallas import tpu_sc as plsc

D_MODEL = 64
SCALE = 8.0

NC = 2
NS = 16
L = 16
NW = NC * NS

IR = 128
G = 4
ROWS_G = G * IR


@functools.lru_cache(maxsize=None)
def _build(B):
    n_ir = B // IR
    ir_per_w = n_ir // NW
    n_groups = ir_per_w // G
    mesh = plsc.VectorSubcoreMesh(core_axis_name="c", subcore_axis_name="s")

    @functools.partial(
        pl.kernel,
        mesh=mesh,
        out_type=jax.ShapeDtypeStruct((B, D_MODEL), jnp.float32),
        scratch_types=[
            pltpu.VMEM((G, IR), jnp.int32),
            pltpu.VMEM((ROWS_G, D_MODEL), jnp.float32),
            pltpu.SemaphoreType.DMA,
        ],
        compiler_params=pltpu.CompilerParams(use_tc_tiling_on_sc=False),
    )
    def k(table_hbm, tok_hbm, out_hbm, idx_v, rows_v, sem):
        wid = lax.axis_index("s") * NC + lax.axis_index("c")
        ir0 = wid * ir_per_w

        def step(g, carry):
            base_ir = ir0 + g * G
            pltpu.sync_copy(tok_hbm.at[pl.ds(base_ir, G)], idx_v)
            cps = [
                pltpu.async_copy(
                    table_hbm.at[idx_v.at[j]],
                    rows_v.at[pl.ds(j * IR, IR)],
                    sem,
                )
                for j in range(G)
            ]
            for cp in cps:
                cp.wait()

            def scale_row(r, c2):
                for c in range(D_MODEL // L):
                    sl = pl.ds(c * L, L)
                    rows_v[r, sl] = rows_v[r, sl] * SCALE
                return c2

            lax.fori_loop(0, ROWS_G, scale_row, 0)
            pltpu.sync_copy(rows_v, out_hbm.at[pl.ds(base_ir * IR, ROWS_G)])
            return carry

        lax.fori_loop(0, n_groups, step, 0)

    return k


@jax.jit
def kernel(tokens, table):
    S, T = tokens.shape
    B = S * T
    tok2 = tokens.reshape(B // IR, IR).astype(jnp.int32)
    out = _build(B)(table, tok2)
    return out.reshape(S, T, D_MODEL)
